# Initial kernel scaffold; baseline (speedup 1.0000x reference)
import jax
import jax.numpy as jnp
from jax.experimental import pallas as pl


def kernel(x, edge_index, pre_W0, pre_b0, bn_g, bn_b, pre_W1, pre_b1, ln1_g, ln1_b, ln2_g, ln2_b, sage_Wself, sage_Wneigh, sage_b, gnnfc_W, gnnfc_b, ffd_W1, ffd_b1, ffd_W2, ffd_b2, pool_W, pool_b, fc1_W, fc1_b, fc2_W, fc2_b):
    raise NotImplementedError("write your pallas kernel here")


# speedup vs baseline: 14.2490x; 14.2490x over previous
import functools

import jax
import jax.numpy as jnp
from jax import lax
from jax.experimental import pallas as pl
from jax.experimental.pallas import tpu as pltpu
from jax.experimental.pallas import tpu_sc as plsc

N = 10000
D = 128
E = 320000
NWORK = 32
EPW = E // NWORK
BATCH = 128
NFULL = EPW // BATCH
REM = EPW - NFULL * BATCH
STRIDE = 624
WIN = 640
NS = 10240


def _ln_rows(x, g, b):
    m = jnp.mean(x, axis=1, keepdims=True)
    v = jnp.mean((x - m) ** 2, axis=1, keepdims=True)
    return (x - m) * lax.rsqrt(v + 1e-5) * g + b


def _build_segsum():
    mesh = plsc.VectorSubcoreMesh(core_axis_name="c", subcore_axis_name="s")

    @functools.partial(
        pl.kernel,
        out_type=[
            jax.ShapeDtypeStruct((N, D), jnp.float32),
            jax.ShapeDtypeStruct((N, D), jnp.float32),
            jax.ShapeDtypeStruct((NS,), jnp.float32),
            jax.ShapeDtypeStruct((NS,), jnp.float32),
            jax.ShapeDtypeStruct((2 * 16 * NS,), jnp.float32),
        ],
        mesh=mesh,
        compiler_params=pltpu.CompilerParams(needs_layout_passes=False),
        scratch_types=[
            pltpu.VMEM((BATCH,), jnp.int32),
            pltpu.VMEM((BATCH,), jnp.int32),
            pltpu.VMEM((BATCH, D), jnp.float32),
            pltpu.VMEM((REM,), jnp.int32),
            pltpu.VMEM((REM,), jnp.int32),
            pltpu.VMEM((REM, D), jnp.float32),
            pltpu.VMEM((N,), jnp.float32),
            pltpu.VMEM((NS,), jnp.float32),
            pltpu.VMEM((16, WIN), jnp.float32),
            pltpu.VMEM((WIN,), jnp.float32),
            pltpu.VMEM_SHARED((N, D), jnp.float32),
            pltpu.SemaphoreType.DMA,
        ],
    )
    def seg(table, mask1, srcs, dsts, zeros, p0, p1, d0, d1, dstage,
            sidx, didx, rows, sidx2, didx2, rows2,
            maskb, degb, redb, sumb, acc, sem):
        cid = lax.axis_index("c")
        sid = lax.axis_index("s")
        wid = sid * 2 + cid
        rbase = sid * STRIDE
        pltpu.sync_copy(zeros.at[pl.ds(rbase, WIN)], acc.at[pl.ds(rbase, WIN)])
        pltpu.sync_copy(mask1, maskb)
        z16 = jnp.zeros((16,), jnp.float32)

        def zb(i, c):
            degb[pl.ds(i * 16, 16)] = z16
            return c

        lax.fori_loop(0, NS // 16, zb, 0)
        plsc.subcore_barrier()
        ebase = wid * EPW

        def dodeg(sbuf, dbuf, nchunk):
            for j2 in range(nchunk):
                sv = sbuf[pl.ds(j2 * 16, 16)]
                dv = dbuf[pl.ds(j2 * 16, 16)]
                mv = plsc.load_gather(maskb, [sv])
                plsc.addupdate_scatter(degb, [dv], mv)

        def body(j, c):
            off = pl.multiple_of(ebase + j * BATCH, 8)
            pltpu.sync_copy(srcs.at[pl.ds(off, BATCH)], sidx)
            pltpu.sync_copy(dsts.at[pl.ds(off, BATCH)], didx)
            pltpu.async_copy(table.at[sidx], rows, sem).wait()
            pltpu.sync_copy(rows, acc.at[didx], add=True)
            dodeg(sidx, didx, BATCH // 16)
            return c

        lax.fori_loop(0, NFULL, body, 0)
        off = pl.multiple_of(ebase + NFULL * BATCH, 8)
        pltpu.sync_copy(srcs.at[pl.ds(off, REM)], sidx2)
        pltpu.sync_copy(dsts.at[pl.ds(off, REM)], didx2)
        pltpu.async_copy(table.at[sidx2], rows2, sem).wait()
        pltpu.sync_copy(rows2, acc.at[didx2], add=True)
        dodeg(sidx2, didx2, REM // 16)
        dbase = sid * WIN
        doff = pl.multiple_of((cid * 16 + sid) * NS, 128)
        pltpu.sync_copy(degb, dstage.at[pl.ds(doff, NS)])
        plsc.subcore_barrier()
        for r in range(16):
            roff = pl.multiple_of((cid * 16 + r) * NS + dbase, 128)
            pltpu.sync_copy(dstage.at[pl.ds(roff, WIN)], redb.at[r])

        def rb(ci, c):
            t = redb[0, pl.ds(ci * 16, 16)]
            for r in range(1, 16):
                t = t + redb[r, pl.ds(ci * 16, 16)]
            sumb[pl.ds(ci * 16, 16)] = t
            return c

        lax.fori_loop(0, WIN // 16, rb, 0)

        @pl.when(cid == 0)
        def _():
            pltpu.sync_copy(acc.at[pl.ds(rbase, WIN)], p0.at[pl.ds(rbase, WIN)])
            pltpu.sync_copy(sumb, d0.at[pl.ds(dbase, WIN)])

        @pl.when(cid == 1)
        def _():
            pltpu.sync_copy(acc.at[pl.ds(rbase, WIN)], p1.at[pl.ds(rbase, WIN)])
            pltpu.sync_copy(sumb, d1.at[pl.ds(dbase, WIN)])

    return seg


def _pre_body(x_ref, w0, b0, bng, bnb, w1, b1, g1, gb1, h_out, tbl_out):
    x = x_ref[...]
    h = jnp.dot(x, w0[...], preferred_element_type=jnp.float32) + b0[...]
    mu = jnp.mean(h, axis=0, keepdims=True)
    var = jnp.mean((h - mu) ** 2, axis=0, keepdims=True)
    h = (h - mu) * lax.rsqrt(var + 1e-5) * bng[...] + bnb[...]
    h = jnp.maximum(h, 0.0)
    h = jnp.dot(h, w1[...], preferred_element_type=jnp.float32) + b1[...]
    h_out[...] = h
    tbl_out[...] = _ln_rows(h, g1[...], gb1[...])


def _run_pre(x, w0, b0, bng, bnb, w1, b1, g1, gb1):
    return pl.pallas_call(
        _pre_body,
        out_shape=[
            jax.ShapeDtypeStruct((N, D), jnp.float32),
            jax.ShapeDtypeStruct((N, D), jnp.float32),
        ],
    )(x, w0, b0, bng, bnb, w1, b1, g1, gb1)


def _blk_body(h_ref, tbl_ref, p0, p1, dg0, dg1, mk, ws0, wn0, ws1, wn1,
              hb0, hb1, gw, gb, lg2, lb2, fw1, fb1, fw2, fb2,
              h_out, ptbl_out):
    h = h_ref[...]
    xnm = tbl_ref[...]
    mask = mk[...]
    deg = mask * (dg0[...] + dg1[...]) + mask
    ssum = mask * (p0[...] + p1[...]) + xnm
    neigh = ssum / jnp.maximum(deg, 1.0)
    hd0 = (jnp.dot(xnm, ws0[...], preferred_element_type=jnp.float32)
           + jnp.dot(neigh, wn0[...], preferred_element_type=jnp.float32)
           + hb0[...])
    hd1 = (jnp.dot(xnm, ws1[...], preferred_element_type=jnp.float32)
           + jnp.dot(neigh, wn1[...], preferred_element_type=jnp.float32)
           + hb1[...])
    cat = jnp.maximum(jnp.concatenate([hd0, hd1], axis=1), 0.0)
    g = jnp.dot(cat, gw[...], preferred_element_type=jnp.float32) + gb[...]
    h = h + g * mask
    xn2 = _ln_rows(h, lg2[...], lb2[...])
    f = jnp.maximum(
        jnp.dot(xn2, fw1[...], preferred_element_type=jnp.float32) + fb1[...],
        0.0)
    f = jnp.dot(f, fw2[...], preferred_element_type=jnp.float32) + fb2[...]
    h = h + f * mask
    h_out[...] = h
    dn = lax.rsqrt(jnp.maximum(deg, 1.0))
    ptbl_out[...] = h * dn * mask


_BLK_ROWS = 2000


def _run_blk(h, tbl, p0, p1, dg0, dg1, mk, ws0, wn0, ws1, wn1, hb0, hb1,
             gw, gb, lg2, lb2, fw1, fb1, fw2, fb2):
    big = lambda: pl.BlockSpec((_BLK_ROWS, D), lambda i: (i, 0))
    one = lambda: pl.BlockSpec((_BLK_ROWS, 1), lambda i: (i, 0))
    w = lambda a: pl.BlockSpec(a.shape, lambda i: (0, 0))
    args = (h, tbl, p0, p1, dg0, dg1, mk, ws0, wn0, ws1, wn1, hb0, hb1,
            gw, gb, lg2, lb2, fw1, fb1, fw2, fb2)
    specs = ([big(), big(), big(), big(), one(), one(), one()]
             + [w(a) for a in args[7:]])
    return pl.pallas_call(
        _blk_body,
        grid=(N // _BLK_ROWS,),
        in_specs=specs,
        out_specs=[big(), big()],
        out_shape=[
            jax.ShapeDtypeStruct((N, D), jnp.float32),
            jax.ShapeDtypeStruct((N, D), jnp.float32),
        ],
    )(*args)


def _topk_keep(score, k):
    u = lax.bitcast_convert_type(score, jnp.uint32)
    neg = u >= jnp.uint32(0x80000000)
    key = jnp.where(neg, jnp.uint32(0xFFFFFFFF) - u,
                    u | jnp.uint32(0x80000000))

    def tbody(b, t):
        bit = (31 - b).astype(jnp.uint32)
        cand = t | lax.shift_left(jnp.uint32(1), bit)
        cnt = jnp.sum(jnp.where(key >= cand, 1, 0))
        return jnp.where(cnt >= k, cand, t)

    t = lax.fori_loop(0, 32, tbody, jnp.uint32(0))
    grt = key > t
    eq = key == t
    r = k - jnp.sum(jnp.where(grt, 1, 0))
    kidx = lax.broadcasted_iota(jnp.int32, score.shape, 0)

    def mbody(b, m):
        cand = m | lax.shift_left(jnp.int32(1), 14 - b)
        cnt = jnp.sum(jnp.where(eq & (kidx < cand), 1, 0))
        return jnp.where(cnt < r, cand, m)

    m = lax.fori_loop(0, 15, mbody, jnp.int32(0))
    return grt | (eq & (kidx <= m))


_NBLK = N // _BLK_ROWS


def _score_body(pt_ref, q0, q1, dg0, dg1, mk, pw, pb, score_out):
    mask = mk[...]
    deg = mask * (dg0[...] + dg1[...]) + mask
    dn = lax.rsqrt(jnp.maximum(deg, 1.0))
    agg = dn * mask * (q0[...] + q1[...] + pt_ref[...])
    sc = jnp.dot(agg, pw[...], preferred_element_type=jnp.float32) + pb[...]
    score_out[...] = jnp.where(mask > 0, sc, -1e30)


def _run_score(pt, q0, q1, dg0, dg1, mk, pw, pb):
    big = lambda: pl.BlockSpec((_BLK_ROWS, D), lambda i: (i, 0))
    one = lambda: pl.BlockSpec((_BLK_ROWS, 1), lambda i: (i, 0))
    w = lambda a: pl.BlockSpec(a.shape, lambda i: (0, 0))
    return pl.pallas_call(
        _score_body,
        grid=(_NBLK,),
        in_specs=[big(), big(), big(), one(), one(), one(), w(pw), w(pb)],
        out_specs=one(),
        out_shape=jax.ShapeDtypeStruct((N, 1), jnp.float32),
    )(pt, q0, q1, dg0, dg1, mk, pw, pb)


def _select_body(score_ref, nm_out, *, k):
    nm_out[...] = _topk_keep(score_ref[...], k).astype(jnp.float32)


def _run_select(score, k):
    return pl.pallas_call(
        functools.partial(_select_body, k=k),
        out_shape=jax.ShapeDtypeStruct((N, 1), jnp.float32),
    )(score)


def _readout(h3, keep):
    bmax = jnp.max(jnp.where(keep, h3, -1e30), axis=0, keepdims=True)
    bsum = jnp.sum(h3, axis=0, keepdims=True)
    return jnp.concatenate([bmax, bsum], axis=1)


def _apply0_body(h_ref, score_ref, nm_ref, lg1, lb1,
                 h_out, tbl_out, pick_out, *, k):
    i = pl.program_id(0)
    newmask = nm_ref[...]
    h3 = h_ref[...] * jnp.tanh(score_ref[...]) * newmask
    h_out[...] = h3
    tbl_out[...] = _ln_rows(h3, lg1[...], lb1[...]) * newmask
    blk = jnp.broadcast_to(_readout(h3, newmask > 0), (8, 2 * D))

    @pl.when(i == 0)
    def _():
        pick_out[...] = blk

    @pl.when(i > 0)
    def _():
        prev = pick_out[...]
        pick_out[...] = jnp.concatenate(
            [jnp.maximum(prev[:, 0:D], blk[:, 0:D]),
             prev[:, D:] + blk[:, D:]], axis=1)

    @pl.when(i == _NBLK - 1)
    def _():
        cur = pick_out[...]
        pick_out[...] = jnp.concatenate(
            [cur[:, 0:D], cur[:, D:] * (1.0 / k)], axis=1)


def _run_apply0(h, score, nm, lg1, lb1, k):
    big = lambda: pl.BlockSpec((_BLK_ROWS, D), lambda i: (i, 0))
    one = lambda: pl.BlockSpec((_BLK_ROWS, 1), lambda i: (i, 0))
    w = lambda a: pl.BlockSpec(a.shape, lambda i: (0, 0))
    return pl.pallas_call(
        functools.partial(_apply0_body, k=k),
        grid=(_NBLK,),
        in_specs=[big(), one(), one(), w(lg1), w(lb1)],
        out_specs=[big(), big(), pl.BlockSpec((8, 2 * D), lambda i: (0, 0))],
        out_shape=[
            jax.ShapeDtypeStruct((N, D), jnp.float32),
            jax.ShapeDtypeStruct((N, D), jnp.float32),
            jax.ShapeDtypeStruct((8, 2 * D), jnp.float32),
        ],
    )(h, score, nm, lg1, lb1)


def _apply1_body(h_ref, score_ref, nm_ref, pick0, fw1, fb1, fw2, fb2,
                 out_ref, acc_ref, *, k):
    i = pl.program_id(0)
    newmask = nm_ref[...]
    h3 = h_ref[...] * jnp.tanh(score_ref[...]) * newmask
    blk = _readout(h3, newmask > 0)

    @pl.when(i == 0)
    def _():
        acc_ref[...] = blk
        out_ref[...] = jnp.zeros((1, 32), jnp.float32)

    @pl.when(i > 0)
    def _():
        prev = acc_ref[...]
        acc_ref[...] = jnp.concatenate(
            [jnp.maximum(prev[:, 0:D], blk[:, 0:D]),
             prev[:, D:] + blk[:, D:]], axis=1)

    @pl.when(i == _NBLK - 1)
    def _():
        cur = acc_ref[...]
        s = jnp.concatenate([cur[:, 0:D], cur[:, D:] * (1.0 / k)], axis=1)
        s = s + pick0[0:1, :]
        o = jnp.maximum(
            jnp.dot(s, fw1[...], preferred_element_type=jnp.float32)
            + fb1[...], 0.0)
        o = jnp.maximum(
            jnp.dot(o, fw2[...], preferred_element_type=jnp.float32)
            + fb2[...], 0.0)
        out_ref[...] = o


def _run_apply1(h, score, nm, pick0, fw1, fb1, fw2, fb2, k):
    big = lambda: pl.BlockSpec((_BLK_ROWS, D), lambda i: (i, 0))
    one = lambda: pl.BlockSpec((_BLK_ROWS, 1), lambda i: (i, 0))
    w = lambda a: pl.BlockSpec(a.shape, lambda i: (0, 0))
    return pl.pallas_call(
        functools.partial(_apply1_body, k=k),
        grid=(_NBLK,),
        in_specs=[big(), one(), one(), w(pick0), w(fw1), w(fb1), w(fw2),
                  w(fb2)],
        out_specs=pl.BlockSpec((1, 32), lambda i: (0, 0)),
        out_shape=jax.ShapeDtypeStruct((1, 32), jnp.float32),
        scratch_shapes=[pltpu.VMEM((1, 2 * D), jnp.float32)],
    )(h, score, nm, pick0, fw1, fb1, fw2, fb2)


def kernel(x, edge_index, pre_W0, pre_b0, bn_g, bn_b, pre_W1, pre_b1,
           ln1_g, ln1_b, ln2_g, ln2_b, sage_Wself, sage_Wneigh, sage_b,
           gnnfc_W, gnnfc_b, ffd_W1, ffd_b1, ffd_W2, ffd_b2,
           pool_W, pool_b, fc1_W, fc1_b, fc2_W, fc2_b):
    src = edge_index[0]
    dst = edge_index[1]
    zeros = jnp.zeros((N, D), jnp.float32)
    row = lambda v: v.reshape(1, -1)
    col = lambda v: v[:N].reshape(N, 1)

    seg = _build_segsum()

    h, tbl = _run_pre(x, pre_W0, row(pre_b0), row(bn_g), row(bn_b),
                      pre_W1, row(pre_b1), row(ln1_g[0]), row(ln1_b[0]))

    mask1 = jnp.ones((N,), jnp.float32)
    mask2 = jnp.ones((N, 1), jnp.float32)
    ks = [N, 5000]
    pick0 = None
    for i in range(2):
        p0, p1, e0, e1, _ = seg(tbl, mask1, src, dst, zeros)
        dg0, dg1 = col(e0), col(e1)
        h, pt = _run_blk(
            h, tbl, p0, p1, dg0, dg1, mask2,
            sage_Wself[i, 0], sage_Wneigh[i, 0],
            sage_Wself[i, 1], sage_Wneigh[i, 1],
            row(sage_b[i, 0]), row(sage_b[i, 1]),
            gnnfc_W[i], row(gnnfc_b[i]),
            row(ln2_g[i]), row(ln2_b[i]),
            ffd_W1[i], row(ffd_b1[i]), ffd_W2[i], row(ffd_b2[i]))
        q0, q1, _, _, _ = seg(pt, mask1, src, dst, zeros)
        k = (ks[i] + 1) // 2
        score = _run_score(pt, q0, q1, dg0, dg1, mask2,
                           pool_W[i].reshape(D, 1), pool_b[i].reshape(1, 1))
        nm = _run_select(score, k)
        if i == 0:
            h, tbl, pick0 = _run_apply0(h, score, nm,
                                        row(ln1_g[1]), row(ln1_b[1]), k)
            mask2 = nm
            mask1 = nm.reshape(N)
        else:
            out = _run_apply1(h, score, nm, pick0,
                              fc1_W, row(fc1_b), fc2_W, row(fc2_b), k)
    return out
